# Initial kernel scaffold; baseline (speedup 1.0000x reference)
import jax
import jax.numpy as jnp
from jax.experimental import pallas as pl


def kernel(input, target):
    raise NotImplementedError("write your pallas kernel here")


# speedup vs baseline: 63.5339x; 63.5339x over previous
import jax
import jax.numpy as jnp
from jax.experimental import pallas as pl

_DELTA_VAR = 0.5
_DELTA_DIST = 1.5
_ALPHA = 1.0
_BETA = 1.0
_GAMMA = 0.001
_NC = 16


def _loss_kernel(x_ref, t_ref, out_ref):
    b = pl.program_id(0)
    nb = pl.num_programs(0)

    x = x_ref[0]
    t = t_ref[0]
    F, P = x.shape

    lbl = jax.lax.broadcasted_iota(jnp.int32, (_NC, 1), 0)
    mask = (t == lbl).astype(jnp.float32)

    counts = jnp.sum(mask, axis=1, keepdims=True)
    safe_counts = jnp.maximum(counts, 1.0)
    present = counts > 0.0

    sums = jax.lax.dot_general(
        mask, x, dimension_numbers=(((1,), (1,)), ((), ())),
        preferred_element_type=jnp.float32)
    means = sums / safe_counts

    c_means = jax.lax.dot_general(
        means, mask, dimension_numbers=(((0,), (0,)), ((), ())),
        preferred_element_type=jnp.float32)

    dev = jnp.sum(jnp.abs(x - c_means), axis=0, keepdims=True)
    var = jnp.maximum(dev - _DELTA_VAR, 0.0) ** 2
    var_sums = jax.lax.dot_general(
        mask, var, dimension_numbers=(((1,), (1,)), ((), ())),
        preferred_element_type=jnp.float32)
    c_var = jnp.where(present, var_sums / safe_counts, 0.0)
    var_term = jnp.sum(c_var)

    mc = c_means[:, :_NC]
    d = jnp.sum(jnp.abs(mc[:, :, None] - mc[:, None, :]), axis=0)
    row = jax.lax.broadcasted_iota(jnp.int32, (_NC, _NC), 0)
    col = jax.lax.broadcasted_iota(jnp.int32, (_NC, _NC), 1)
    off_diag = (row != col).astype(jnp.float32)
    margin = 2.0 * _DELTA_DIST * off_diag
    presf = present.astype(jnp.float32)
    pair_mask = presf * presf.T
    c_dist = jnp.sum(pair_mask * jnp.maximum(margin - d, 0.0) ** 2)
    K = jnp.sum(presf)
    denom = jnp.maximum(K * (K - 1.0), 1.0)
    dist_term = jnp.where(K > 1.0, c_dist / denom, 0.0)

    col_norms = jnp.where(present, jnp.sum(jnp.abs(means), axis=1,
                                           keepdims=True), 0.0)
    reg_term = jnp.sum(col_norms) / K

    contrib = (_ALPHA * var_term + _BETA * dist_term +
               _GAMMA * reg_term) / nb

    @pl.when(b == 0)
    def _():
        out_ref[...] = jnp.zeros((1, 1), jnp.float32)

    out_ref[...] += jnp.full((1, 1), contrib, jnp.float32)


def kernel(input, target):
    B, F, P = input.shape
    t3 = target.reshape(B, 1, P)
    out = pl.pallas_call(
        _loss_kernel,
        grid=(B,),
        in_specs=[
            pl.BlockSpec((1, F, P), lambda i: (i, 0, 0)),
            pl.BlockSpec((1, 1, P), lambda i: (i, 0, 0)),
        ],
        out_specs=pl.BlockSpec((1, 1), lambda i: (0, 0)),
        out_shape=jax.ShapeDtypeStruct((1, 1), jnp.float32),
    )(input, t3)
    return out[0, 0]


# speedup vs baseline: 66.1066x; 1.0405x over previous
import jax
import jax.numpy as jnp
import numpy as np
from jax.experimental import pallas as pl

_DELTA_VAR = 0.5
_DELTA_DIST = 1.5
_ALPHA = 1.0
_BETA = 1.0
_GAMMA = 0.001
_NC = 16


def _batch_contrib(x, t, rj, tk, mg, ones_p):
    lbl = jax.lax.broadcasted_iota(jnp.int32, (_NC, 1), 0)
    mask = (t == lbl).astype(jnp.float32)

    counts = jax.lax.dot_general(
        mask, ones_p, dimension_numbers=(((1,), (1,)), ((), ())),
        preferred_element_type=jnp.float32)
    safe_counts = jnp.maximum(counts, 1.0)
    present = counts > 0.0

    sums = jax.lax.dot_general(
        mask, x, dimension_numbers=(((1,), (1,)), ((), ())),
        preferred_element_type=jnp.float32)
    means = sums / safe_counts

    c_means = jax.lax.dot_general(
        means, mask, dimension_numbers=(((0,), (0,)), ((), ())),
        preferred_element_type=jnp.float32)

    dev = jnp.sum(jnp.abs(x - c_means), axis=0, keepdims=True)
    var = jnp.maximum(dev - _DELTA_VAR, 0.0) ** 2
    var_sums = jax.lax.dot_general(
        mask, var, dimension_numbers=(((1,), (1,)), ((), ())),
        preferred_element_type=jnp.float32)
    c_var = jnp.where(present, var_sums / safe_counts, 0.0)
    var_term = jnp.sum(c_var)

    mc = c_means[:, :_NC]
    mc_j = jax.lax.dot_general(
        mc, rj, dimension_numbers=(((1,), (0,)), ((), ())),
        preferred_element_type=jnp.float32)
    mc_k = jax.lax.dot_general(
        mc, tk, dimension_numbers=(((1,), (0,)), ((), ())),
        preferred_element_type=jnp.float32)
    d = jnp.sum(jnp.abs(mc_j - mc_k), axis=0, keepdims=True)
    presf = present.astype(jnp.float32)
    pres_j = jax.lax.dot_general(
        presf, rj, dimension_numbers=(((0,), (0,)), ((), ())),
        preferred_element_type=jnp.float32)
    pres_k = jax.lax.dot_general(
        presf, tk, dimension_numbers=(((0,), (0,)), ((), ())),
        preferred_element_type=jnp.float32)
    hinge = jnp.maximum(mg - d, 0.0) ** 2
    c_dist = jnp.sum(pres_j * pres_k * hinge)
    K = jnp.sum(presf)
    denom = jnp.maximum(K * (K - 1.0), 1.0)
    dist_term = jnp.where(K > 1.0, c_dist / denom, 0.0)

    col_norms = jnp.where(present, jnp.sum(jnp.abs(means), axis=1,
                                           keepdims=True), 0.0)
    reg_term = jnp.sum(col_norms) / K

    return (_ALPHA * var_term + _BETA * dist_term + _GAMMA * reg_term)


def _loss_kernel(x_ref, t_ref, rj_ref, tk_ref, mg_ref, out_ref):
    b = pl.program_id(0)
    nb = pl.num_programs(0)
    bpb = x_ref.shape[0]
    P = x_ref.shape[2]

    rj = rj_ref[...]
    tk = tk_ref[...]
    mg = mg_ref[...]
    ones_p = jnp.ones((1, P), jnp.float32)

    contrib = 0.0
    for bb in range(bpb):
        contrib = contrib + _batch_contrib(
            x_ref[bb], t_ref[bb], rj, tk, mg, ones_p)
    contrib = contrib / (nb * bpb)

    @pl.when(b == 0)
    def _():
        out_ref[...] = jnp.zeros((1, 1), jnp.float32)

    out_ref[...] += jnp.full((1, 1), contrib, jnp.float32)


def _pair_constants():
    nc = _NC
    rj = np.zeros((nc, nc * nc), np.float32)
    tk = np.zeros((nc, nc * nc), np.float32)
    for j in range(nc):
        for k in range(nc):
            rj[j, j * nc + k] = 1.0
            tk[k, j * nc + k] = 1.0
    mg = np.full((1, nc * nc), 2.0 * _DELTA_DIST, np.float32)
    for j in range(nc):
        mg[0, j * nc + j] = 0.0
    return jnp.asarray(rj), jnp.asarray(tk), jnp.asarray(mg)


def kernel(input, target):
    B, F, P = input.shape
    t3 = target.reshape(B, 1, P)
    rj, tk, mg = _pair_constants()
    nn = _NC * _NC
    bpb = 2 if B % 2 == 0 else 1
    out = pl.pallas_call(
        _loss_kernel,
        grid=(B // bpb,),
        in_specs=[
            pl.BlockSpec((bpb, F, P), lambda i: (i, 0, 0)),
            pl.BlockSpec((bpb, 1, P), lambda i: (i, 0, 0)),
            pl.BlockSpec((_NC, nn), lambda i: (0, 0)),
            pl.BlockSpec((_NC, nn), lambda i: (0, 0)),
            pl.BlockSpec((1, nn), lambda i: (0, 0)),
        ],
        out_specs=pl.BlockSpec((1, 1), lambda i: (0, 0)),
        out_shape=jax.ShapeDtypeStruct((1, 1), jnp.float32),
    )(input, t3, rj, tk, mg)
    return out[0, 0]
